# baseline (speedup 1.0000x reference)
import functools

import jax
import jax.numpy as jnp
from jax import lax
from jax.experimental import pallas as pl
from jax.experimental.pallas import tpu as pltpu

NUM_GRAPHS_C = 256
H_C = 4
D_C = 128
ITEM_PAD = 512
WIN = 32


def _fused_body(x_ref, gf_ref, pure_ref,
                ig_ref, ilo_ref, ihi_ref, is_ref, istart_ref,
                w_ref, b_ref, b4_ref,
                hg_ref, wout_ref, wk_ref, *, blk, r):
    nb_groups = blk // r
    i = pl.program_id(0)

    x = x_ref[...]
    logits = jax.lax.dot_general(
        x, w_ref[...], (((1,), (1,)), ((), ())),
        preferred_element_type=jnp.float32,
        precision=jax.lax.Precision.HIGHEST)
    w = jnp.tanh(logits + b_ref[...])
    wout_ref[...] = w

    wb = jax.lax.dot_general(
        w.astype(jnp.bfloat16), b4_ref[...],
        (((1,), (0,)), ((), ())),
        preferred_element_type=jnp.float32)
    weighted = jnp.concatenate(
        [x * wb[:, h * D_C:(h + 1) * D_C] for h in range(H_C)], axis=1)
    wk_ref[...] = weighted

    grp = wk_ref[...].reshape(nb_groups, r, H_C * D_C).sum(axis=1)

    gf = gf_ref[0]
    pure = pure_ref[0]
    iota = lax.broadcasted_iota(jnp.int32, (NUM_GRAPHS_C, nb_groups), 0)
    onehot = jnp.where(gf == iota, pure, 0.0)
    contrib = jnp.dot(onehot, grp,
                      preferred_element_type=jnp.float32,
                      precision=jax.lax.Precision.HIGHEST)

    @pl.when(i == 0)
    def _():
        hg_ref[...] = jnp.zeros_like(hg_ref)

    hg_ref[...] += contrib

    riota = lax.broadcasted_iota(jnp.int32, (WIN, 1), 0)

    def item_body(j, _):
        g = ig_ref[j] - i * nb_groups
        base = g * r
        base8 = (base // 8) * 8
        off = base - base8
        lo = off + ilo_ref[j]
        hi = off + ihi_ref[j]
        m = jnp.where((riota >= lo) & (riota < hi), 1.0, 0.0)
        win = wk_ref[pl.ds(base8, WIN), :]
        piece = jnp.sum(win * m, axis=0, keepdims=True)
        hg_ref[pl.ds(is_ref[j], 1), :] += piece
        return 0

    lax.fori_loop(istart_ref[i], istart_ref[i + 1], item_body, 0)


@jax.jit
def kernel(edge_feats, segment_ids, W, b):
    e, d = edge_feats.shape
    h = W.shape[0]
    blk = 1600
    r = 32
    nb = e // blk
    nb_groups = blk // r
    ng = e // r

    seg_first = segment_ids[::r]
    seg_last = segment_ids[r - 1::r]
    imp_mask = seg_first != seg_last
    gf = seg_first.reshape(nb, 1, nb_groups)
    pure = (~imp_mask).astype(jnp.float32).reshape(nb, 1, nb_groups)

    bounds = jnp.searchsorted(
        segment_ids, jnp.arange(NUM_GRAPHS_C + 1, dtype=jnp.int32))
    st = bounds[:-1].astype(jnp.int32)
    en = bounds[1:].astype(jnp.int32)
    nonempty = en > st
    en1 = jnp.maximum(en - 1, 0)
    g1 = st // r
    g2 = en1 // r
    single = g1 == g2
    imp_g1 = imp_mask[jnp.minimum(g1, ng - 1)]
    imp_g2 = imp_mask[jnp.minimum(g2, ng - 1)]
    valid_a = nonempty & imp_g1
    valid_b = nonempty & (~single) & imp_g2
    ia_g = jnp.where(valid_a, g1, ng)
    ia_lo = st % r
    ia_hi = jnp.where(single, en1 % r + 1, r)
    ib_g = jnp.where(valid_b, g2, ng)
    ib_lo = jnp.zeros_like(st)
    ib_hi = en1 % r + 1
    segs = jnp.arange(NUM_GRAPHS_C, dtype=jnp.int32)
    item_g = jnp.stack([ia_g, ib_g], axis=1).reshape(-1)
    item_lo = jnp.stack([ia_lo, ib_lo], axis=1).reshape(-1)
    item_hi = jnp.stack([ia_hi, ib_hi], axis=1).reshape(-1)
    item_s = jnp.stack([segs, segs], axis=1).reshape(-1)
    order = jnp.argsort(item_g)
    item_g = item_g[order].astype(jnp.int32)
    item_lo = item_lo[order].astype(jnp.int32)
    item_hi = item_hi[order].astype(jnp.int32)
    item_s = item_s[order].astype(jnp.int32)
    istart = jnp.searchsorted(
        item_g, jnp.arange(nb + 1, dtype=jnp.int32) * nb_groups
    ).astype(jnp.int32)

    b2 = b.reshape(1, h)
    b4 = jnp.repeat(jnp.eye(h, dtype=jnp.bfloat16), d, axis=1)

    def smem1d(n):
        return pl.BlockSpec(memory_space=pltpu.SMEM, block_shape=(n,),
                            index_map=lambda i: (0,))

    grid_spec = pltpu.PrefetchScalarGridSpec(
        num_scalar_prefetch=0,
        grid=(nb,),
        in_specs=[
            pl.BlockSpec((blk, d), lambda i: (i, 0)),
            pl.BlockSpec((1, 1, nb_groups), lambda i: (i, 0, 0)),
            pl.BlockSpec((1, 1, nb_groups), lambda i: (i, 0, 0)),
            smem1d(ITEM_PAD),
            smem1d(ITEM_PAD),
            smem1d(ITEM_PAD),
            smem1d(ITEM_PAD),
            smem1d(nb + 1),
            pl.BlockSpec((h, d), lambda i: (0, 0)),
            pl.BlockSpec((1, h), lambda i: (0, 0)),
            pl.BlockSpec((h, h * d), lambda i: (0, 0)),
        ],
        out_specs=[
            pl.BlockSpec((NUM_GRAPHS_C, H_C * D_C), lambda i: (0, 0)),
            pl.BlockSpec((blk, h), lambda i: (i, 0)),
        ],
        scratch_shapes=[pltpu.VMEM((blk, H_C * D_C), jnp.float32)],
    )

    hg, weights = pl.pallas_call(
        functools.partial(_fused_body, blk=blk, r=r),
        grid_spec=grid_spec,
        out_shape=[
            jax.ShapeDtypeStruct((NUM_GRAPHS_C, H_C * D_C), jnp.float32),
            jax.ShapeDtypeStruct((e, h), jnp.float32),
        ],
    )(edge_feats, gf, pure,
      item_g, item_lo, item_hi, item_s, istart, W, b2, b4)
    return hg, weights


# speedup vs baseline: 1.2267x; 1.2267x over previous
import functools

import jax
import jax.numpy as jnp
from jax import lax
from jax.experimental import pallas as pl
from jax.experimental.pallas import tpu as pltpu
from jax.experimental.pallas import tpu_sc as plsc

NUM_GRAPHS_C = 256
H_C = 4
D_C = 128
HD = H_C * D_C
ITEM_PAD = 512
WIN = 32

BLK = 2560
R = 32
NB_GROUPS = BLK // R

NS = 16
NG_PAD = 10240
GP_TILE = NG_PAD // NS
CH = 64
NCH = GP_TILE // CH
IP_TILE = ITEM_PAD // NS


def _tc_body(x_ref, gf_ref, pure_ref,
             ig_ref, ilo_ref, ihi_ref, istart_ref,
             w_ref, b_ref, b4_ref,
             wout_ref, g_ref, p_ref, wk_ref):
    i = pl.program_id(0)

    x = x_ref[...]
    logits = jax.lax.dot_general(
        x, w_ref[...], (((1,), (1,)), ((), ())),
        preferred_element_type=jnp.float32,
        precision=jax.lax.Precision.HIGHEST)
    w = jnp.tanh(logits + b_ref[...])
    wout_ref[...] = w

    wb = jax.lax.dot_general(
        w.astype(jnp.bfloat16), b4_ref[...],
        (((1,), (0,)), ((), ())),
        preferred_element_type=jnp.float32)
    weighted = jnp.concatenate(
        [x * wb[:, h * D_C:(h + 1) * D_C] for h in range(H_C)], axis=1)
    wk_ref[...] = weighted

    g_ref[...] = wk_ref[...].reshape(NB_GROUPS, R, HD).sum(axis=1)

    @pl.when(i == 0)
    def _():
        p_ref[...] = jnp.zeros_like(p_ref)

    riota = lax.broadcasted_iota(jnp.int32, (WIN, 1), 0)

    def item_body(j, _):
        g = ig_ref[j] - i * NB_GROUPS
        lo = ilo_ref[j]
        hi = ihi_ref[j]
        m = jnp.where((riota >= lo) & (riota < hi), 1.0, 0.0)
        win = wk_ref[pl.ds(g * R, WIN), :]
        p_ref[pl.ds(j, 1), :] = jnp.sum(win * m, axis=0, keepdims=True)
        return 0

    lax.fori_loop(istart_ref[i], istart_ref[i + 1], item_body, 0)


def _sc_body(g_hbm, p_hbm, gidx_hbm, pidx_hbm, out_hbm, dump_hbm,
             idx_v, g_v, pidx_v, p_v, acc_v, outb_v, mbuf_v):
    c = lax.axis_index("c")
    s = lax.axis_index("s")
    col0 = c * 256
    z16 = jnp.zeros((16,), jnp.float32)

    def zrow(i, _):
        def zcol(k, _):
            acc_v[i, pl.ds(k * 16, 16)] = z16
            return 0
        lax.fori_loop(0, 16, zcol, 0)
        return 0
    lax.fori_loop(0, NUM_GRAPHS_C + 1, zrow, 0)

    def add_rows(src_v, tgt_v, q):
        for rr in range(16):
            t_row = tgt_v[rr]
            for k in range(16):
                acc_v[t_row, pl.ds(k * 16, 16)] += (
                    src_v[q * 16 + rr, pl.ds(k * 16, 16)])

    def chunk_body(ch, _):
        pltpu.sync_copy(gidx_hbm.at[s, pl.ds(ch * CH, CH)], idx_v.at[ch])
        row0 = s * GP_TILE + ch * CH
        pltpu.sync_copy(g_hbm.at[pl.ds(row0, CH), pl.ds(col0, 256)], g_v)

        def qloop(q, _):
            tgt = idx_v[ch, pl.ds(q * 16, 16)]
            add_rows(g_v, tgt, q)
            return 0
        lax.fori_loop(0, CH // 16, qloop, 0)
        return 0
    lax.fori_loop(0, NCH, chunk_body, 0)

    pltpu.sync_copy(pidx_hbm.at[s], pidx_v.at[0])
    pltpu.sync_copy(p_hbm.at[pl.ds(s * IP_TILE, IP_TILE), pl.ds(col0, 256)],
                    p_v)

    def pqloop(q, _):
        tgt = pidx_v[0, pl.ds(q * 16, 16)]
        add_rows(p_v, tgt, q)
        return 0
    lax.fori_loop(0, IP_TILE // 16, pqloop, 0)

    pltpu.sync_copy(acc_v, dump_hbm.at[s, :, pl.ds(col0, 256)])
    plsc.subcore_barrier()

    def zob(i, _):
        def zcol(k, _):
            outb_v[i, pl.ds(k * 16, 16)] = z16
            return 0
        lax.fori_loop(0, 16, zcol, 0)
        return 0
    lax.fori_loop(0, 16, zob, 0)

    def merge_src(src, _):
        pltpu.sync_copy(
            dump_hbm.at[src, pl.ds(s * 16, 16), pl.ds(col0, 256)], mbuf_v)
        def mrow(i, _):
            def mcol(k, _):
                outb_v[i, pl.ds(k * 16, 16)] += mbuf_v[i, pl.ds(k * 16, 16)]
                return 0
            lax.fori_loop(0, 16, mcol, 0)
            return 0
        lax.fori_loop(0, 16, mrow, 0)
        return 0
    lax.fori_loop(0, NS, merge_src, 0)

    pltpu.sync_copy(outb_v, out_hbm.at[pl.ds(s * 16, 16), pl.ds(col0, 256)])


@jax.jit
def kernel(edge_feats, segment_ids, W, b):
    e, d = edge_feats.shape
    h = W.shape[0]
    nb = e // BLK
    ng = e // R

    seg_first = segment_ids[::R]
    seg_last = segment_ids[R - 1::R]
    imp_mask = seg_first != seg_last
    gf = seg_first.reshape(nb, 1, NB_GROUPS)
    pure = (~imp_mask).astype(jnp.float32).reshape(nb, 1, NB_GROUPS)

    gidx = jnp.where(imp_mask, NUM_GRAPHS_C, seg_first).astype(jnp.int32)
    gidx = jnp.concatenate(
        [gidx, jnp.full((NG_PAD - ng,), NUM_GRAPHS_C, jnp.int32)])

    bounds = jnp.searchsorted(
        segment_ids, jnp.arange(NUM_GRAPHS_C + 1, dtype=jnp.int32))
    st = bounds[:-1].astype(jnp.int32)
    en = bounds[1:].astype(jnp.int32)
    nonempty = en > st
    en1 = jnp.maximum(en - 1, 0)
    g1 = st // R
    g2 = en1 // R
    single = g1 == g2
    imp_g1 = imp_mask[jnp.minimum(g1, ng - 1)]
    imp_g2 = imp_mask[jnp.minimum(g2, ng - 1)]
    valid_a = nonempty & imp_g1
    valid_b = nonempty & (~single) & imp_g2
    ia_g = jnp.where(valid_a, g1, ng)
    ia_lo = st % R
    ia_hi = jnp.where(single, en1 % R + 1, R)
    ib_g = jnp.where(valid_b, g2, ng)
    ib_lo = jnp.zeros_like(st)
    ib_hi = en1 % R + 1
    segs = jnp.arange(NUM_GRAPHS_C, dtype=jnp.int32)
    item_g = jnp.stack([ia_g, ib_g], axis=1).reshape(-1)
    item_lo = jnp.stack([ia_lo, ib_lo], axis=1).reshape(-1)
    item_hi = jnp.stack([ia_hi, ib_hi], axis=1).reshape(-1)
    item_s = jnp.stack([segs, segs], axis=1).reshape(-1)
    valid = jnp.stack([valid_a, valid_b], axis=1).reshape(-1)
    order = jnp.argsort(item_g)
    item_g = item_g[order].astype(jnp.int32)
    item_lo = item_lo[order].astype(jnp.int32)
    item_hi = item_hi[order].astype(jnp.int32)
    pidx = jnp.where(valid[order], item_s[order],
                     NUM_GRAPHS_C).astype(jnp.int32)
    istart = jnp.searchsorted(
        item_g, jnp.arange(nb + 1, dtype=jnp.int32) * NB_GROUPS
    ).astype(jnp.int32)

    b2 = b.reshape(1, h)
    b4 = jnp.repeat(jnp.eye(h, dtype=jnp.bfloat16), d, axis=1)

    def smem1d(n):
        return pl.BlockSpec(memory_space=pltpu.SMEM, block_shape=(n,),
                            index_map=lambda i: (0,))

    grid_spec = pltpu.PrefetchScalarGridSpec(
        num_scalar_prefetch=0,
        grid=(nb,),
        in_specs=[
            pl.BlockSpec((BLK, d), lambda i: (i, 0)),
            pl.BlockSpec((1, 1, NB_GROUPS), lambda i: (i, 0, 0)),
            pl.BlockSpec((1, 1, NB_GROUPS), lambda i: (i, 0, 0)),
            smem1d(ITEM_PAD),
            smem1d(ITEM_PAD),
            smem1d(ITEM_PAD),
            smem1d(nb + 1),
            pl.BlockSpec((h, d), lambda i: (0, 0)),
            pl.BlockSpec((1, h), lambda i: (0, 0)),
            pl.BlockSpec((h, HD), lambda i: (0, 0)),
        ],
        out_specs=[
            pl.BlockSpec((BLK, h), lambda i: (i, 0)),
            pl.BlockSpec((NB_GROUPS, HD), lambda i: (i, 0)),
            pl.BlockSpec((ITEM_PAD, HD), lambda i: (0, 0)),
        ],
        scratch_shapes=[pltpu.VMEM((BLK, HD), jnp.float32)],
    )

    weights, g_sums, pieces = pl.pallas_call(
        _tc_body,
        grid_spec=grid_spec,
        out_shape=[
            jax.ShapeDtypeStruct((e, h), jnp.float32),
            jax.ShapeDtypeStruct((NG_PAD, HD), jnp.float32),
            jax.ShapeDtypeStruct((ITEM_PAD, HD), jnp.float32),
        ],
    )(edge_feats, gf, pure, item_g, item_lo, item_hi, istart, W, b2, b4)

    mesh = plsc.VectorSubcoreMesh(core_axis_name="c", subcore_axis_name="s")
    hg, _dump = pl.kernel(
        _sc_body,
        mesh=mesh,
        out_type=[
            jax.ShapeDtypeStruct((NUM_GRAPHS_C, HD), jnp.float32),
            jax.ShapeDtypeStruct((NS, NUM_GRAPHS_C + 1, HD), jnp.float32),
        ],
        scratch_types=[
            pltpu.VMEM((NCH, CH), jnp.int32),
            pltpu.VMEM((CH, 256), jnp.float32),
            pltpu.VMEM((1, IP_TILE), jnp.int32),
            pltpu.VMEM((IP_TILE, 256), jnp.float32),
            pltpu.VMEM((NUM_GRAPHS_C + 1, 256), jnp.float32),
            pltpu.VMEM((16, 256), jnp.float32),
            pltpu.VMEM((16, 256), jnp.float32),
        ],
    )(g_sums, pieces, gidx.reshape(NS, GP_TILE), pidx.reshape(NS, IP_TILE))

    return hg, weights


# speedup vs baseline: 1.8761x; 1.5294x over previous
import functools

import jax
import jax.numpy as jnp
from jax import lax
from jax.experimental import pallas as pl
from jax.experimental.pallas import tpu as pltpu
from jax.experimental.pallas import tpu_sc as plsc

NUM_GRAPHS_C = 256
H_C = 4
D_C = 128
HD = H_C * D_C
ITEM_PAD = 512
WIN = 32

BLK = 6400
R = 32
NB_GROUPS = BLK // R

NS = 16
NG_PAD = 10240
GP_TILE = NG_PAD // NS
CH = 64
NCH = GP_TILE // CH
IP_TILE = ITEM_PAD // NS


def _tc_body(x_ref, gf_ref, pure_ref,
             ig_ref, ilo_ref, ihi_ref, istart_ref,
             w_ref, b_ref, b4_ref,
             wout_ref, g_ref, p_ref, wk_ref):
    i = pl.program_id(0)

    x = x_ref[...]
    logits = jax.lax.dot_general(
        x.astype(jnp.bfloat16), w_ref[...].astype(jnp.bfloat16),
        (((1,), (1,)), ((), ())),
        preferred_element_type=jnp.float32)
    w = jnp.tanh(logits + b_ref[...])
    wout_ref[...] = w

    wb = jax.lax.dot_general(
        w.astype(jnp.bfloat16), b4_ref[...],
        (((1,), (0,)), ((), ())),
        preferred_element_type=jnp.float32)
    weighted = jnp.concatenate(
        [x * wb[:, h * D_C:(h + 1) * D_C] for h in range(H_C)], axis=1)
    wk_ref[...] = weighted

    g_ref[...] = wk_ref[...].reshape(NB_GROUPS, R, HD).sum(axis=1)

    @pl.when(i == 0)
    def _():
        p_ref[...] = jnp.zeros_like(p_ref)

    riota = lax.broadcasted_iota(jnp.int32, (WIN, 1), 0)

    def item_body(j, _):
        g = ig_ref[j] - i * NB_GROUPS
        lo = ilo_ref[j]
        hi = ihi_ref[j]
        m = jnp.where((riota >= lo) & (riota < hi), 1.0, 0.0)
        win = wk_ref[pl.ds(g * R, WIN), :]
        p_ref[pl.ds(j, 1), :] = jnp.sum(win * m, axis=0, keepdims=True)
        return 0

    lax.fori_loop(istart_ref[i], istart_ref[i + 1], item_body, 0)


def _sc_body(g_hbm, p_hbm, gidx_hbm, pidx_hbm, out_hbm, dump_hbm,
             idx_v, g_v, pidx_v, p_v, acc_v, outb_v, mbuf_v):
    c = lax.axis_index("c")
    s = lax.axis_index("s")
    col0 = c * 256
    z16 = jnp.zeros((16,), jnp.float32)

    def zrow(i, _):
        def zcol(k, _):
            acc_v[i, pl.ds(k * 16, 16)] = z16
            return 0
        lax.fori_loop(0, 16, zcol, 0)
        return 0
    lax.fori_loop(0, NUM_GRAPHS_C + 1, zrow, 0)

    def add_rows(src_v, tgt_v, q):
        for rr in range(16):
            t_row = tgt_v[rr]
            for k in range(16):
                acc_v[t_row, pl.ds(k * 16, 16)] += (
                    src_v[q * 16 + rr, pl.ds(k * 16, 16)])

    def chunk_body(ch, _):
        pltpu.sync_copy(gidx_hbm.at[s, pl.ds(ch * CH, CH)], idx_v.at[ch])
        row0 = s * GP_TILE + ch * CH
        pltpu.sync_copy(g_hbm.at[pl.ds(row0, CH), pl.ds(col0, 256)], g_v)

        def qloop(q, _):
            tgt = idx_v[ch, pl.ds(q * 16, 16)]
            add_rows(g_v, tgt, q)
            return 0
        lax.fori_loop(0, CH // 16, qloop, 0)
        return 0
    lax.fori_loop(0, NCH, chunk_body, 0)

    pltpu.sync_copy(pidx_hbm.at[s], pidx_v.at[0])
    pltpu.sync_copy(p_hbm.at[pl.ds(s * IP_TILE, IP_TILE), pl.ds(col0, 256)],
                    p_v)

    def pqloop(q, _):
        tgt = pidx_v[0, pl.ds(q * 16, 16)]
        add_rows(p_v, tgt, q)
        return 0
    lax.fori_loop(0, IP_TILE // 16, pqloop, 0)

    pltpu.sync_copy(acc_v, dump_hbm.at[s, :, pl.ds(col0, 256)])
    plsc.subcore_barrier()

    def zob(i, _):
        def zcol(k, _):
            outb_v[i, pl.ds(k * 16, 16)] = z16
            return 0
        lax.fori_loop(0, 16, zcol, 0)
        return 0
    lax.fori_loop(0, 16, zob, 0)

    def merge_src(src, _):
        pltpu.sync_copy(
            dump_hbm.at[src, pl.ds(s * 16, 16), pl.ds(col0, 256)], mbuf_v)
        def mrow(i, _):
            def mcol(k, _):
                outb_v[i, pl.ds(k * 16, 16)] += mbuf_v[i, pl.ds(k * 16, 16)]
                return 0
            lax.fori_loop(0, 16, mcol, 0)
            return 0
        lax.fori_loop(0, 16, mrow, 0)
        return 0
    lax.fori_loop(0, NS, merge_src, 0)

    pltpu.sync_copy(outb_v, out_hbm.at[pl.ds(s * 16, 16), pl.ds(col0, 256)])


@jax.jit
def kernel(edge_feats, segment_ids, W, b):
    e, d = edge_feats.shape
    h = W.shape[0]
    nb = e // BLK
    ng = e // R

    seg_first = segment_ids[::R]
    seg_last = segment_ids[R - 1::R]
    imp_mask = seg_first != seg_last
    gf = seg_first.reshape(nb, 1, NB_GROUPS)
    pure = (~imp_mask).astype(jnp.float32).reshape(nb, 1, NB_GROUPS)

    gidx = jnp.where(imp_mask, NUM_GRAPHS_C, seg_first).astype(jnp.int32)
    gidx = jnp.concatenate(
        [gidx, jnp.full((NG_PAD - ng,), NUM_GRAPHS_C, jnp.int32)])

    bounds = jnp.searchsorted(
        segment_ids, jnp.arange(NUM_GRAPHS_C + 1, dtype=jnp.int32))
    st = bounds[:-1].astype(jnp.int32)
    en = bounds[1:].astype(jnp.int32)
    nonempty = en > st
    en1 = jnp.maximum(en - 1, 0)
    g1 = st // R
    g2 = en1 // R
    single = g1 == g2
    imp_g1 = imp_mask[jnp.minimum(g1, ng - 1)]
    imp_g2 = imp_mask[jnp.minimum(g2, ng - 1)]
    valid_a = nonempty & imp_g1
    valid_b = nonempty & (~single) & imp_g2
    ia_g = jnp.where(valid_a, g1, ng)
    ia_lo = st % R
    ia_hi = jnp.where(single, en1 % R + 1, R)
    ib_g = jnp.where(valid_b, g2, ng)
    ib_lo = jnp.zeros_like(st)
    ib_hi = en1 % R + 1
    segs = jnp.arange(NUM_GRAPHS_C, dtype=jnp.int32)
    item_g = jnp.stack([ia_g, ib_g], axis=1).reshape(-1)
    item_lo = jnp.stack([ia_lo, ib_lo], axis=1).reshape(-1)
    item_hi = jnp.stack([ia_hi, ib_hi], axis=1).reshape(-1)
    item_s = jnp.stack([segs, segs], axis=1).reshape(-1)
    valid = jnp.stack([valid_a, valid_b], axis=1).reshape(-1)
    order = jnp.argsort(item_g)
    item_g = item_g[order].astype(jnp.int32)
    item_lo = item_lo[order].astype(jnp.int32)
    item_hi = item_hi[order].astype(jnp.int32)
    pidx = jnp.where(valid[order], item_s[order],
                     NUM_GRAPHS_C).astype(jnp.int32)
    istart = jnp.searchsorted(
        item_g, jnp.arange(nb + 1, dtype=jnp.int32) * NB_GROUPS
    ).astype(jnp.int32)

    b2 = b.reshape(1, h)
    b4 = jnp.repeat(jnp.eye(h, dtype=jnp.bfloat16), d, axis=1)

    def smem1d(n):
        return pl.BlockSpec(memory_space=pltpu.SMEM, block_shape=(n,),
                            index_map=lambda i: (0,))

    grid_spec = pltpu.PrefetchScalarGridSpec(
        num_scalar_prefetch=0,
        grid=(nb,),
        in_specs=[
            pl.BlockSpec((BLK, d), lambda i: (i, 0)),
            pl.BlockSpec((1, 1, NB_GROUPS), lambda i: (i, 0, 0)),
            pl.BlockSpec((1, 1, NB_GROUPS), lambda i: (i, 0, 0)),
            smem1d(ITEM_PAD),
            smem1d(ITEM_PAD),
            smem1d(ITEM_PAD),
            smem1d(nb + 1),
            pl.BlockSpec((h, d), lambda i: (0, 0)),
            pl.BlockSpec((1, h), lambda i: (0, 0)),
            pl.BlockSpec((h, HD), lambda i: (0, 0)),
        ],
        out_specs=[
            pl.BlockSpec((BLK, h), lambda i: (i, 0)),
            pl.BlockSpec((NB_GROUPS, HD), lambda i: (i, 0)),
            pl.BlockSpec((ITEM_PAD, HD), lambda i: (0, 0)),
        ],
        scratch_shapes=[pltpu.VMEM((BLK, HD), jnp.float32)],
    )

    weights, g_sums, pieces = pl.pallas_call(
        _tc_body,
        grid_spec=grid_spec,
        out_shape=[
            jax.ShapeDtypeStruct((e, h), jnp.float32),
            jax.ShapeDtypeStruct((NG_PAD, HD), jnp.float32),
            jax.ShapeDtypeStruct((ITEM_PAD, HD), jnp.float32),
        ],
    )(edge_feats, gf, pure, item_g, item_lo, item_hi, istart, W, b2, b4)

    mesh = plsc.VectorSubcoreMesh(core_axis_name="c", subcore_axis_name="s")
    hg, _dump = pl.kernel(
        _sc_body,
        mesh=mesh,
        out_type=[
            jax.ShapeDtypeStruct((NUM_GRAPHS_C, HD), jnp.float32),
            jax.ShapeDtypeStruct((NS, NUM_GRAPHS_C + 1, HD), jnp.float32),
        ],
        scratch_types=[
            pltpu.VMEM((NCH, CH), jnp.int32),
            pltpu.VMEM((CH, 256), jnp.float32),
            pltpu.VMEM((1, IP_TILE), jnp.int32),
            pltpu.VMEM((IP_TILE, 256), jnp.float32),
            pltpu.VMEM((NUM_GRAPHS_C + 1, 256), jnp.float32),
            pltpu.VMEM((16, 256), jnp.float32),
            pltpu.VMEM((16, 256), jnp.float32),
        ],
    )(g_sums, pieces, gidx.reshape(NS, GP_TILE), pidx.reshape(NS, IP_TILE))

    return hg, weights


# speedup vs baseline: 1.9416x; 1.0349x over previous
import functools

import jax
import jax.numpy as jnp
from jax import lax
from jax.experimental import pallas as pl
from jax.experimental.pallas import tpu as pltpu
from jax.experimental.pallas import tpu_sc as plsc

NUM_GRAPHS_C = 256
H_C = 4
D_C = 128
HD = H_C * D_C
ITEM_PAD = 512
WIN = 32

BLK = 12800
R = 32
NB_GROUPS = BLK // R

NS = 16
NG_PAD = 10240
GP_TILE = NG_PAD // NS
CH = 64
NCH = GP_TILE // CH
IP_TILE = ITEM_PAD // NS


def _tc_body(x_ref, gf_ref, pure_ref,
             ig_ref, ilo_ref, ihi_ref, istart_ref,
             w_ref, b_ref, b4_ref,
             wout_ref, g_ref, p_ref, wk_ref):
    i = pl.program_id(0)

    x = x_ref[...]
    logits = jax.lax.dot_general(
        x.astype(jnp.bfloat16), w_ref[...].astype(jnp.bfloat16),
        (((1,), (1,)), ((), ())),
        preferred_element_type=jnp.float32)
    w = jnp.tanh(logits + b_ref[...])
    wout_ref[...] = w

    wb = jax.lax.dot_general(
        w.astype(jnp.bfloat16), b4_ref[...],
        (((1,), (0,)), ((), ())),
        preferred_element_type=jnp.float32)
    weighted = jnp.concatenate(
        [x * wb[:, h * D_C:(h + 1) * D_C] for h in range(H_C)], axis=1)
    wk_ref[...] = weighted

    g_ref[...] = wk_ref[...].reshape(NB_GROUPS, R, HD).sum(axis=1)

    @pl.when(i == 0)
    def _():
        p_ref[...] = jnp.zeros_like(p_ref)

    riota = lax.broadcasted_iota(jnp.int32, (WIN, 1), 0)

    def item_body(j, _):
        g = ig_ref[j] - i * NB_GROUPS
        lo = ilo_ref[j]
        hi = ihi_ref[j]
        m = jnp.where((riota >= lo) & (riota < hi), 1.0, 0.0)
        win = wk_ref[pl.ds(g * R, WIN), :]
        p_ref[pl.ds(j, 1), :] = jnp.sum(win * m, axis=0, keepdims=True)
        return 0

    lax.fori_loop(istart_ref[i], istart_ref[i + 1], item_body, 0)


def _sc_body(g_hbm, p_hbm, gidx_hbm, pidx_hbm, out_hbm, dump_hbm,
             idx_v, g_v, pidx_v, p_v, acc_v, outb_v, mbuf_v):
    c = lax.axis_index("c")
    s = lax.axis_index("s")
    col0 = c * 256
    z16 = jnp.zeros((16,), jnp.float32)

    def zrow(i, _):
        def zcol(k, _):
            acc_v[i, pl.ds(k * 16, 16)] = z16
            return 0
        lax.fori_loop(0, 16, zcol, 0)
        return 0
    lax.fori_loop(0, NUM_GRAPHS_C + 1, zrow, 0)

    def add_rows(src_v, tgt_v, q):
        for rr in range(16):
            t_row = tgt_v[rr]
            for k in range(16):
                acc_v[t_row, pl.ds(k * 16, 16)] += (
                    src_v[q * 16 + rr, pl.ds(k * 16, 16)])

    def chunk_body(ch, _):
        pltpu.sync_copy(gidx_hbm.at[s, pl.ds(ch * CH, CH)], idx_v.at[ch])
        row0 = s * GP_TILE + ch * CH
        pltpu.sync_copy(g_hbm.at[pl.ds(row0, CH), pl.ds(col0, 256)], g_v)

        def qloop(q, _):
            tgt = idx_v[ch, pl.ds(q * 16, 16)]
            add_rows(g_v, tgt, q)
            return 0
        lax.fori_loop(0, CH // 16, qloop, 0)
        return 0
    lax.fori_loop(0, NCH, chunk_body, 0)

    pltpu.sync_copy(pidx_hbm.at[s], pidx_v.at[0])
    pltpu.sync_copy(p_hbm.at[pl.ds(s * IP_TILE, IP_TILE), pl.ds(col0, 256)],
                    p_v)

    def pqloop(q, _):
        tgt = pidx_v[0, pl.ds(q * 16, 16)]
        add_rows(p_v, tgt, q)
        return 0
    lax.fori_loop(0, IP_TILE // 16, pqloop, 0)

    pltpu.sync_copy(acc_v, dump_hbm.at[s, :, pl.ds(col0, 256)])
    plsc.subcore_barrier()

    def zob(i, _):
        def zcol(k, _):
            outb_v[i, pl.ds(k * 16, 16)] = z16
            return 0
        lax.fori_loop(0, 16, zcol, 0)
        return 0
    lax.fori_loop(0, 16, zob, 0)

    def merge_src(src, _):
        pltpu.sync_copy(
            dump_hbm.at[src, pl.ds(s * 16, 16), pl.ds(col0, 256)], mbuf_v)
        def mrow(i, _):
            def mcol(k, _):
                outb_v[i, pl.ds(k * 16, 16)] += mbuf_v[i, pl.ds(k * 16, 16)]
                return 0
            lax.fori_loop(0, 16, mcol, 0)
            return 0
        lax.fori_loop(0, 16, mrow, 0)
        return 0
    lax.fori_loop(0, NS, merge_src, 0)

    pltpu.sync_copy(outb_v, out_hbm.at[pl.ds(s * 16, 16), pl.ds(col0, 256)])


@jax.jit
def kernel(edge_feats, segment_ids, W, b):
    e, d = edge_feats.shape
    h = W.shape[0]
    nb = e // BLK
    ng = e // R

    seg_first = segment_ids[::R]
    seg_last = segment_ids[R - 1::R]
    imp_mask = seg_first != seg_last
    gf = seg_first.reshape(nb, 1, NB_GROUPS)
    pure = (~imp_mask).astype(jnp.float32).reshape(nb, 1, NB_GROUPS)

    gidx = jnp.where(imp_mask, NUM_GRAPHS_C, seg_first).astype(jnp.int32)
    gidx = jnp.concatenate(
        [gidx, jnp.full((NG_PAD - ng,), NUM_GRAPHS_C, jnp.int32)])

    bounds = jnp.searchsorted(
        segment_ids, jnp.arange(NUM_GRAPHS_C + 1, dtype=jnp.int32))
    st = bounds[:-1].astype(jnp.int32)
    en = bounds[1:].astype(jnp.int32)
    nonempty = en > st
    en1 = jnp.maximum(en - 1, 0)
    g1 = st // R
    g2 = en1 // R
    single = g1 == g2
    imp_g1 = imp_mask[jnp.minimum(g1, ng - 1)]
    imp_g2 = imp_mask[jnp.minimum(g2, ng - 1)]
    valid_a = nonempty & imp_g1
    valid_b = nonempty & (~single) & imp_g2
    ia_g = jnp.where(valid_a, g1, ng)
    ia_lo = st % R
    ia_hi = jnp.where(single, en1 % R + 1, R)
    ib_g = jnp.where(valid_b, g2, ng)
    ib_lo = jnp.zeros_like(st)
    ib_hi = en1 % R + 1
    segs = jnp.arange(NUM_GRAPHS_C, dtype=jnp.int32)
    item_g = jnp.stack([ia_g, ib_g], axis=1).reshape(-1)
    item_lo = jnp.stack([ia_lo, ib_lo], axis=1).reshape(-1)
    item_hi = jnp.stack([ia_hi, ib_hi], axis=1).reshape(-1)
    item_s = jnp.stack([segs, segs], axis=1).reshape(-1)
    valid = jnp.stack([valid_a, valid_b], axis=1).reshape(-1)
    order = jnp.argsort(item_g)
    item_g = item_g[order].astype(jnp.int32)
    item_lo = item_lo[order].astype(jnp.int32)
    item_hi = item_hi[order].astype(jnp.int32)
    pidx = jnp.where(valid[order], item_s[order],
                     NUM_GRAPHS_C).astype(jnp.int32)
    istart = jnp.searchsorted(
        item_g, jnp.arange(nb + 1, dtype=jnp.int32) * NB_GROUPS
    ).astype(jnp.int32)

    b2 = b.reshape(1, h)
    b4 = jnp.repeat(jnp.eye(h, dtype=jnp.bfloat16), d, axis=1)

    def smem1d(n):
        return pl.BlockSpec(memory_space=pltpu.SMEM, block_shape=(n,),
                            index_map=lambda i: (0,))

    grid_spec = pltpu.PrefetchScalarGridSpec(
        num_scalar_prefetch=0,
        grid=(nb,),
        in_specs=[
            pl.BlockSpec((BLK, d), lambda i: (i, 0)),
            pl.BlockSpec((1, 1, NB_GROUPS), lambda i: (i, 0, 0)),
            pl.BlockSpec((1, 1, NB_GROUPS), lambda i: (i, 0, 0)),
            smem1d(ITEM_PAD),
            smem1d(ITEM_PAD),
            smem1d(ITEM_PAD),
            smem1d(nb + 1),
            pl.BlockSpec((h, d), lambda i: (0, 0)),
            pl.BlockSpec((1, h), lambda i: (0, 0)),
            pl.BlockSpec((h, HD), lambda i: (0, 0)),
        ],
        out_specs=[
            pl.BlockSpec((BLK, h), lambda i: (i, 0)),
            pl.BlockSpec((NB_GROUPS, HD), lambda i: (i, 0)),
            pl.BlockSpec((ITEM_PAD, HD), lambda i: (0, 0)),
        ],
        scratch_shapes=[pltpu.VMEM((BLK, HD), jnp.float32)],
    )

    weights, g_sums, pieces = pl.pallas_call(
        _tc_body,
        grid_spec=grid_spec,
        out_shape=[
            jax.ShapeDtypeStruct((e, h), jnp.float32),
            jax.ShapeDtypeStruct((NG_PAD, HD), jnp.float32),
            jax.ShapeDtypeStruct((ITEM_PAD, HD), jnp.float32),
        ],
    )(edge_feats, gf, pure, item_g, item_lo, item_hi, istart, W, b2, b4)

    mesh = plsc.VectorSubcoreMesh(core_axis_name="c", subcore_axis_name="s")
    hg, _dump = pl.kernel(
        _sc_body,
        mesh=mesh,
        out_type=[
            jax.ShapeDtypeStruct((NUM_GRAPHS_C, HD), jnp.float32),
            jax.ShapeDtypeStruct((NS, NUM_GRAPHS_C + 1, HD), jnp.float32),
        ],
        scratch_types=[
            pltpu.VMEM((NCH, CH), jnp.int32),
            pltpu.VMEM((CH, 256), jnp.float32),
            pltpu.VMEM((1, IP_TILE), jnp.int32),
            pltpu.VMEM((IP_TILE, 256), jnp.float32),
            pltpu.VMEM((NUM_GRAPHS_C + 1, 256), jnp.float32),
            pltpu.VMEM((16, 256), jnp.float32),
            pltpu.VMEM((16, 256), jnp.float32),
        ],
    )(g_sums, pieces, gidx.reshape(NS, GP_TILE), pidx.reshape(NS, IP_TILE))

    return hg, weights
